# baseline (speedup 1.0000x reference)
import jax
import jax.numpy as jnp
from jax import lax
from jax.experimental import pallas as pl
from jax.experimental.pallas import tpu as pltpu
from jax.experimental.pallas import tpu_sc as plsc

B = 16384
D = 64
NC = 2
NS = 16
NW = NC * NS
BPW = B // NW


def _body(midx_hbm, uidx_hbm, wm_hbm, wu_hbm, mu_hbm, bm_hbm, bu_hbm, out_hbm,
          midx_v, uidx_v, rows_m, rows_u, bm_v, bu_v, mu_v, out_v,
          sem_m, sem_u, sem_bm, sem_bu):
    wid = lax.axis_index("s") * NC + lax.axis_index("c")
    base = wid * BPW

    pltpu.sync_copy(midx_hbm.at[pl.ds(base, BPW)], midx_v)
    pltpu.sync_copy(uidx_hbm.at[pl.ds(base, BPW)], uidx_v)
    pltpu.sync_copy(mu_hbm, mu_v.at[pl.ds(0, 1)])

    cm = pltpu.async_copy(wm_hbm.at[midx_v], rows_m, sem_m)
    cu = pltpu.async_copy(wu_hbm.at[uidx_v], rows_u, sem_u)
    cbm = pltpu.async_copy(bm_hbm.at[midx_v], bm_v, sem_bm)
    cbu = pltpu.async_copy(bu_hbm.at[uidx_v], bu_v, sem_bu)
    cm.wait()
    cu.wait()
    cbm.wait()
    cbu.wait()

    mu_s = mu_v[...][0]
    iota16 = lax.iota(jnp.int32, 16)

    def blk(j, _):
        rb = j * 16
        row_ids = rb + iota16
        accs = [bm_v[pl.ds(rb, 16)] + bu_v[pl.ds(rb, 16)] + mu_s,
                jnp.zeros((16,), jnp.float32),
                jnp.zeros((16,), jnp.float32),
                jnp.zeros((16,), jnp.float32)]
        for d in range(D):
            col = jnp.full((16,), d, jnp.int32)
            a = plsc.load_gather(rows_m, [row_ids, col])
            b = plsc.load_gather(rows_u, [row_ids, col])
            accs[d % 4] = accs[d % 4] + a * b
        out_v[pl.ds(rb, 16)] = (accs[0] + accs[1]) + (accs[2] + accs[3])
        return 0

    lax.fori_loop(0, BPW // 16, blk, 0)

    pltpu.sync_copy(out_v, out_hbm.at[pl.ds(base, BPW)])


@jax.jit
def kernel(x, W_movie, W_user, mu, b_movie, b_user):
    movie_idx = x[:, 1]
    user_idx = x[:, 0]
    mesh = plsc.VectorSubcoreMesh(core_axis_name="c", subcore_axis_name="s",
                                  num_cores=NC, num_subcores=NS)
    run = pl.kernel(
        _body,
        out_type=jax.ShapeDtypeStruct((B,), jnp.float32),
        mesh=mesh,
        compiler_params=pltpu.CompilerParams(needs_layout_passes=False,
                                             use_tc_tiling_on_sc=False),
        scratch_types=[
            pltpu.VMEM((BPW,), jnp.int32),
            pltpu.VMEM((BPW,), jnp.int32),
            pltpu.VMEM((BPW, D), jnp.float32),
            pltpu.VMEM((BPW, D), jnp.float32),
            pltpu.VMEM((BPW,), jnp.float32),
            pltpu.VMEM((BPW,), jnp.float32),
            pltpu.VMEM((16,), jnp.float32),
            pltpu.VMEM((BPW,), jnp.float32),
            pltpu.SemaphoreType.DMA,
            pltpu.SemaphoreType.DMA,
            pltpu.SemaphoreType.DMA,
            pltpu.SemaphoreType.DMA,
        ],
    )
    return run(movie_idx, user_idx, W_movie, W_user,
               mu.reshape(-1), b_movie.reshape(-1), b_user.reshape(-1))


# speedup vs baseline: 1.1827x; 1.1827x over previous
import jax
import jax.numpy as jnp
from jax import lax
from jax.experimental import pallas as pl
from jax.experimental.pallas import tpu as pltpu
from jax.experimental.pallas import tpu_sc as plsc

B = 16384
D = 64
NC = 2
NS = 16
NW = NC * NS
BPW = B // NW
PP = 17


def _body(midx_hbm, uidx_hbm, wm_hbm, wu_hbm, mu_hbm, bm_hbm, bu_hbm, out_hbm,
          midx_v, uidx_v, rows_m, rows_u, bm_v, bu_v, mu_v, part_v,
          out_v, sem_m, sem_u, sem_bm, sem_bu):
    wid = lax.axis_index("s") * NC + lax.axis_index("c")
    base = wid * BPW

    pltpu.sync_copy(midx_hbm.at[pl.ds(base, BPW)], midx_v)
    pltpu.sync_copy(uidx_hbm.at[pl.ds(base, BPW)], uidx_v)
    pltpu.sync_copy(mu_hbm, mu_v.at[pl.ds(0, 1)])

    iota16 = lax.iota(jnp.int32, 16)
    zero16 = jnp.zeros((16,), jnp.int32)

    cm = pltpu.async_copy(wm_hbm.at[midx_v], rows_m, sem_m)
    cu = pltpu.async_copy(wu_hbm.at[uidx_v], rows_u, sem_u)
    cbm = pltpu.async_copy(bm_hbm.at[midx_v], bm_v, sem_bm)
    cbu = pltpu.async_copy(bu_hbm.at[uidx_v], bu_v, sem_bu)
    cm.wait()
    cu.wait()
    cbm.wait()
    cbu.wait()

    def row(r, _):
        a0 = rows_m[r, pl.ds(0, 16)] * rows_u[r, pl.ds(0, 16)]
        a1 = rows_m[r, pl.ds(16, 16)] * rows_u[r, pl.ds(16, 16)]
        a2 = rows_m[r, pl.ds(32, 16)] * rows_u[r, pl.ds(32, 16)]
        a3 = rows_m[r, pl.ds(48, 16)] * rows_u[r, pl.ds(48, 16)]
        part_v[r, pl.ds(0, 16)] = (a0 + a1) + (a2 + a3)
        return 0

    lax.fori_loop(0, BPW, row, 0, unroll=8)

    mu_s = mu_v[...][0]

    def blk(j, _):
        rb = j * 16
        rows = rb + iota16
        acc0 = bm_v[pl.ds(rb, 16)]
        acc1 = bu_v[pl.ds(rb, 16)]
        acc2 = mu_s + plsc.load_gather(part_v, [rows, zero16])
        acc3 = plsc.load_gather(part_v, [rows, zero16 + 1])
        for c in range(2, 16, 2):
            acc2 = acc2 + plsc.load_gather(part_v, [rows, zero16 + c])
            acc3 = acc3 + plsc.load_gather(part_v, [rows, zero16 + (c + 1)])
        out_v[pl.ds(rb, 16)] = (acc0 + acc1) + (acc2 + acc3)
        return 0

    lax.fori_loop(0, BPW // 16, blk, 0, unroll=2)

    pltpu.sync_copy(out_v, out_hbm.at[pl.ds(base, BPW)])


@jax.jit
def kernel(x, W_movie, W_user, mu, b_movie, b_user):
    mesh = plsc.VectorSubcoreMesh(core_axis_name="c", subcore_axis_name="s",
                                  num_cores=NC, num_subcores=NS)
    run = pl.kernel(
        _body,
        out_type=jax.ShapeDtypeStruct((B,), jnp.float32),
        mesh=mesh,
        compiler_params=pltpu.CompilerParams(needs_layout_passes=False,
                                             use_tc_tiling_on_sc=False),
        scratch_types=[
            pltpu.VMEM((BPW,), jnp.int32),
            pltpu.VMEM((BPW,), jnp.int32),
            pltpu.VMEM((BPW, D), jnp.float32),
            pltpu.VMEM((BPW, D), jnp.float32),
            pltpu.VMEM((BPW,), jnp.float32),
            pltpu.VMEM((BPW,), jnp.float32),
            pltpu.VMEM((16,), jnp.float32),
            pltpu.VMEM((BPW, PP), jnp.float32),
            pltpu.VMEM((BPW,), jnp.float32),
            pltpu.SemaphoreType.DMA,
            pltpu.SemaphoreType.DMA,
            pltpu.SemaphoreType.DMA,
            pltpu.SemaphoreType.DMA,
        ],
    )
    return run(x[:, 1], x[:, 0], W_movie, W_user, mu.reshape(-1),
               b_movie.reshape(-1), b_user.reshape(-1))
